# baseline (speedup 1.0000x reference)
import functools

import jax
import jax.numpy as jnp
from jax import lax
from jax.experimental import pallas as pl
from jax.experimental.pallas import tpu as pltpu
from jax.experimental.pallas import tpu_sc as plsc

N = 10000
E = 320000
D = 128
NCH = 10

NCORES = 2
NSUB = 16
NW = NCORES * NSUB
L = 16

ES = E // NW
F = 4
FB = D // F
NTASK = NCH * FB
TT = NTASK // NW
CE = 8000
NCHUNK = E // CE
GC = CE // L
UNROLL = 4

_mesh = plsc.VectorSubcoreMesh(core_axis_name="c", subcore_axis_name="s")
_sc_params = pltpu.CompilerParams(needs_layout_passes=False)


def _wid():
    return lax.axis_index("s") * NCORES + lax.axis_index("c")


@functools.partial(
    pl.kernel,
    out_type=jax.ShapeDtypeStruct((NW, N), jnp.float32),
    mesh=_mesh,
    compiler_params=_sc_params,
    scratch_types=[
        pltpu.VMEM((N,), jnp.float32),
        pltpu.VMEM((ES,), jnp.int32),
        pltpu.VMEM((ES,), jnp.float32),
    ],
)
def _deg_kernel(col_hbm, ew_hbm, deg_out, deg_v, col_v, ew_v):
    wid = _wid()
    base = wid * ES
    pltpu.sync_copy(col_hbm.at[pl.ds(base, ES)], col_v)
    pltpu.sync_copy(ew_hbm.at[pl.ds(base, ES)], ew_v)

    def zbody(i, _):
        deg_v[pl.ds(i * L, L)] = jnp.zeros((L,), jnp.float32)
        return 0

    lax.fori_loop(0, N // L, zbody, 0)

    def ebody(i, _):
        sl = pl.ds(i * L, L)
        plsc.addupdate_scatter(deg_v, [col_v[sl]], ew_v[sl])
        return 0

    lax.fori_loop(0, ES // L, ebody, 0)
    pltpu.sync_copy(deg_v, deg_out.at[wid])


def _dis_body(degp_ref, dis_ref, dissq_ref):
    deg = jnp.sum(degp_ref[...], axis=0) + 1.0
    dis = jnp.where(deg > 0, lax.rsqrt(jnp.maximum(deg, 1e-12)), 0.0)
    dis_ref[...] = dis
    dissq_ref[...] = dis * dis


def _dis_call(deg_part):
    return pl.pallas_call(
        _dis_body,
        out_shape=(
            jax.ShapeDtypeStruct((N,), jnp.float32),
            jax.ShapeDtypeStruct((N,), jnp.float32),
        ),
    )(deg_part)


def _mm_body(wt_ref, x_ref, ht_ref):
    ht_ref[0] = lax.dot_general(
        wt_ref[0], x_ref[0],
        dimension_numbers=(((1,), (1,)), ((), ())),
        preferred_element_type=jnp.float32,
    )


def _mm_call(WT, X):
    return pl.pallas_call(
        _mm_body,
        grid=(NCH,),
        in_specs=[
            pl.BlockSpec((1, D, D), lambda i: (i, 0, 0)),
            pl.BlockSpec((1, N, D), lambda i: (i, 0, 0)),
        ],
        out_specs=pl.BlockSpec((1, D, N), lambda i: (i, 0, 0)),
        out_shape=jax.ShapeDtypeStruct((NCH, D, N), jnp.float32),
    )(WT, X)


@functools.partial(
    pl.kernel,
    out_type=jax.ShapeDtypeStruct((E,), jnp.float32),
    mesh=_mesh,
    compiler_params=_sc_params,
    scratch_types=[
        pltpu.VMEM((N,), jnp.float32),
        pltpu.VMEM((ES,), jnp.int32),
        pltpu.VMEM((ES,), jnp.int32),
        pltpu.VMEM((ES,), jnp.float32),
        pltpu.VMEM((ES,), jnp.float32),
    ],
)
def _norm_kernel(row_hbm, col_hbm, ew_hbm, dis_hbm, norm_out,
                 dis_v, row_v, col_v, ew_v, norm_v):
    wid = _wid()
    base = wid * ES
    pltpu.sync_copy(dis_hbm, dis_v)
    pltpu.sync_copy(row_hbm.at[pl.ds(base, ES)], row_v)
    pltpu.sync_copy(col_hbm.at[pl.ds(base, ES)], col_v)
    pltpu.sync_copy(ew_hbm.at[pl.ds(base, ES)], ew_v)

    def body(i, _):
        sl = pl.ds(i * L, L)
        dr = plsc.load_gather(dis_v, [row_v[sl]])
        dc = plsc.load_gather(dis_v, [col_v[sl]])
        norm_v[sl] = dr * ew_v[sl] * dc
        return 0

    lax.fori_loop(0, ES // L, body, 0)
    pltpu.sync_copy(norm_v, norm_out.at[pl.ds(base, ES)])


@functools.partial(
    pl.kernel,
    out_type=jax.ShapeDtypeStruct((NCH, D, N), jnp.float32),
    mesh=_mesh,
    compiler_params=_sc_params,
    scratch_types=(
        [pltpu.VMEM((N,), jnp.float32)]
        + [pltpu.VMEM((N,), jnp.float32)] * F
        + [pltpu.VMEM((N,), jnp.float32)] * F
        + [
            pltpu.VMEM((CE,), jnp.int32),
            pltpu.VMEM((CE,), jnp.int32),
            pltpu.VMEM((CE,), jnp.float32),
        ]
    ),
)
def _msg_kernel(ht_hbm, row_hbm, col_hbm, norm_hbm, dissq_hbm, mt_out,
                dissq_v, ht0, ht1, ht2, ht3, ac0, ac1, ac2, ac3,
                row_c, col_c, norm_c):
    hts = [ht0, ht1, ht2, ht3]
    accs = [ac0, ac1, ac2, ac3]
    wid = _wid()
    pltpu.sync_copy(dissq_hbm, dissq_v)

    def task(ti, _):
        t = wid * TT + ti
        c = t // FB
        j0 = (t % FB) * F
        for jj in range(F):
            pltpu.sync_copy(ht_hbm.at[c, j0 + jj], hts[jj])

        @plsc.parallel_loop(0, N // L, unroll=4)
        def init(k):
            sl = pl.ds(k * L, L)
            dv = dissq_v[sl]
            for jj in range(F):
                accs[jj][sl] = dv * hts[jj][sl]

        def chunk(ch, _):
            cb = ch * CE
            pltpu.sync_copy(row_hbm.at[pl.ds(cb, CE)], row_c)
            pltpu.sync_copy(col_hbm.at[pl.ds(cb, CE)], col_c)
            pltpu.sync_copy(norm_hbm.at[pl.ds(cb, CE)], norm_c)

            def group(g, _):
                for u in range(UNROLL):
                    sl = pl.ds((g * UNROLL + u) * L, L)
                    rv = row_c[sl]
                    cv = col_c[sl]
                    nv = norm_c[sl]
                    for jj in range(F):
                        hv = plsc.load_gather(hts[jj], [rv])
                        plsc.addupdate_scatter(accs[jj], [cv], hv * nv)
                return 0

            lax.fori_loop(0, GC // UNROLL, group, 0)
            return 0

        lax.fori_loop(0, NCHUNK, chunk, 0)
        for jj in range(F):
            pltpu.sync_copy(accs[jj], mt_out.at[c, j0 + jj])
        return 0

    lax.fori_loop(0, TT, task, 0)


def _fin_body(mt_ref, b_ref, out_ref):
    mt = mt_ref[0]
    ii = lax.broadcasted_iota(jnp.int32, (D, D), 0)
    jj = lax.broadcasted_iota(jnp.int32, (D, D), 1)
    eye = jnp.where(ii == jj, 1.0, 0.0).astype(jnp.float32)
    out = lax.dot_general(
        mt, eye,
        dimension_numbers=(((0,), (0,)), ((), ())),
        preferred_element_type=jnp.float32,
    )
    out_ref[0] = out + b_ref[pl.program_id(0)][None, :]


def _fin_call(MT, b):
    return pl.pallas_call(
        _fin_body,
        grid=(NCH,),
        in_specs=[
            pl.BlockSpec((1, D, N), lambda i: (i, 0, 0)),
            pl.BlockSpec((NCH, D), lambda i: (0, 0)),
        ],
        out_specs=pl.BlockSpec((1, N, D), lambda i: (i, 0, 0)),
        out_shape=jax.ShapeDtypeStruct((NCH, N, D), jnp.float32),
    )(MT, b)


def kernel(x0, x1, x2, x3, x4, x5, x6, x7, x8, x9, edge_index, edge_weight, W, b):
    X = jnp.stack([x0, x1, x2, x3, x4, x5, x6, x7, x8, x9])
    row = edge_index[0]
    col = edge_index[1]
    WT = jnp.swapaxes(W, 1, 2)

    deg_part = _deg_kernel(col, edge_weight)
    dis, dissq = _dis_call(deg_part)
    HT = _mm_call(WT, X)
    norm = _norm_kernel(row, col, edge_weight, dis)
    MT = _msg_kernel(HT, row, col, norm, dissq)
    OUT = _fin_call(MT, b)
    return tuple(OUT[i] for i in range(NCH))


# speedup vs baseline: 4.3108x; 4.3108x over previous
import functools

import jax
import jax.numpy as jnp
from jax import lax
from jax.experimental import pallas as pl
from jax.experimental.pallas import tpu as pltpu
from jax.experimental.pallas import tpu_sc as plsc

N = 10000
E = 320000
D = 128
NCH = 10

NCORES = 2
NSUB = 16
NW = NCORES * NSUB
L = 16
CPS = NCH // NCORES

ES = E // NW
ES2 = E // NSUB
CB = 80
NCHK = ES2 // CB
NROWS = 624
NREM = N - NROWS * NSUB

_mesh = plsc.VectorSubcoreMesh(core_axis_name="c", subcore_axis_name="s")
_sc_params = pltpu.CompilerParams(needs_layout_passes=False)


def _wid():
    return lax.axis_index("s") * NCORES + lax.axis_index("c")


@functools.partial(
    pl.kernel,
    out_type=jax.ShapeDtypeStruct((NW, N), jnp.float32),
    mesh=_mesh,
    compiler_params=_sc_params,
    scratch_types=[
        pltpu.VMEM((N,), jnp.float32),
        pltpu.VMEM((ES,), jnp.int32),
        pltpu.VMEM((ES,), jnp.float32),
    ],
)
def _deg_kernel(col_hbm, ew_hbm, deg_out, deg_v, col_v, ew_v):
    wid = _wid()
    base = wid * ES
    pltpu.sync_copy(col_hbm.at[pl.ds(base, ES)], col_v)
    pltpu.sync_copy(ew_hbm.at[pl.ds(base, ES)], ew_v)

    @plsc.parallel_loop(0, N // L, unroll=4)
    def zbody(i):
        deg_v[pl.ds(i * L, L)] = jnp.zeros((L,), jnp.float32)

    def ebody(i, _):
        sl = pl.ds(i * L, L)
        plsc.addupdate_scatter(deg_v, [col_v[sl]], ew_v[sl])
        return 0

    lax.fori_loop(0, ES // L, ebody, 0)
    pltpu.sync_copy(deg_v, deg_out.at[wid])


def _dis_body(degp_ref, dis_ref):
    deg = jnp.sum(degp_ref[...], axis=0) + 1.0
    dis_ref[...] = jnp.where(deg > 0, lax.rsqrt(jnp.maximum(deg, 1e-12)), 0.0)


def _dis_call(deg_part):
    return pl.pallas_call(
        _dis_body,
        out_shape=jax.ShapeDtypeStruct((N,), jnp.float32),
    )(deg_part)


def _prep_body(x_ref, w_ref, dis2_ref, b_ref, h_ref, sl_ref):
    h = lax.dot_general(
        x_ref[0], w_ref[0],
        dimension_numbers=(((1,), (0,)), ((), ())),
        preferred_element_type=jnp.float32,
    )
    h_ref[0] = h
    d2 = dis2_ref[...] * dis2_ref[...]
    sl_ref[0] = d2 * h + b_ref[pl.program_id(0)][None, :]


def _prep_call(X, W, dis2, b):
    return pl.pallas_call(
        _prep_body,
        grid=(NCH,),
        in_specs=[
            pl.BlockSpec((1, N, D), lambda i: (i, 0, 0)),
            pl.BlockSpec((1, D, D), lambda i: (i, 0, 0)),
            pl.BlockSpec((N, 1), lambda i: (0, 0)),
            pl.BlockSpec((NCH, D), lambda i: (0, 0)),
        ],
        out_specs=(
            pl.BlockSpec((1, N, D), lambda i: (i, 0, 0)),
            pl.BlockSpec((1, N, D), lambda i: (i, 0, 0)),
        ),
        out_shape=(
            jax.ShapeDtypeStruct((NCH, N, D), jnp.float32),
            jax.ShapeDtypeStruct((NCH, N, D), jnp.float32),
        ),
    )(X, W, dis2, b)


@functools.partial(
    pl.kernel,
    out_type=jax.ShapeDtypeStruct((E,), jnp.float32),
    mesh=_mesh,
    compiler_params=_sc_params,
    scratch_types=[
        pltpu.VMEM((N,), jnp.float32),
        pltpu.VMEM((ES,), jnp.int32),
        pltpu.VMEM((ES,), jnp.int32),
        pltpu.VMEM((ES,), jnp.float32),
        pltpu.VMEM((ES,), jnp.float32),
    ],
)
def _norm_kernel(row_hbm, col_hbm, ew_hbm, dis_hbm, norm_out,
                 dis_v, row_v, col_v, ew_v, norm_v):
    wid = _wid()
    base = wid * ES
    pltpu.sync_copy(dis_hbm, dis_v)
    pltpu.sync_copy(row_hbm.at[pl.ds(base, ES)], row_v)
    pltpu.sync_copy(col_hbm.at[pl.ds(base, ES)], col_v)
    pltpu.sync_copy(ew_hbm.at[pl.ds(base, ES)], ew_v)

    def body(i, _):
        sl = pl.ds(i * L, L)
        dr = plsc.load_gather(dis_v, [row_v[sl]])
        dc = plsc.load_gather(dis_v, [col_v[sl]])
        norm_v[sl] = dr * ew_v[sl] * dc
        return 0

    lax.fori_loop(0, ES // L, body, 0)
    pltpu.sync_copy(norm_v, norm_out.at[pl.ds(base, ES)])


def _rep_body(n_ref, o_ref):
    o_ref[...] = n_ref[...] * jnp.ones((1, L), jnp.float32)


def _rep_call(norm2):
    blk = E // 40
    return pl.pallas_call(
        _rep_body,
        grid=(40,),
        in_specs=[pl.BlockSpec((blk, 1), lambda i: (i, 0))],
        out_specs=pl.BlockSpec((blk, L), lambda i: (i, 0)),
        out_shape=jax.ShapeDtypeStruct((E, L), jnp.float32),
    )(norm2)


def _radj_body(r_ref, o_ref):
    o_ref[0] = r_ref[...] + pl.program_id(0) * N


def _radj_call(row2):
    nr = E // D
    return pl.pallas_call(
        _radj_body,
        grid=(NCH,),
        in_specs=[pl.BlockSpec((nr, D), lambda i: (0, 0))],
        out_specs=pl.BlockSpec((1, nr, D), lambda i: (i, 0, 0)),
        out_shape=jax.ShapeDtypeStruct((NCH, nr, D), jnp.int32),
    )(row2)


@functools.partial(
    pl.kernel,
    out_type=jax.ShapeDtypeStruct((NCH, N, D), jnp.float32),
    mesh=_mesh,
    compiler_params=_sc_params,
    scratch_types=(
        [
            pltpu.VMEM((4, CB), jnp.int32),
            pltpu.VMEM((4, CB), jnp.int32),
            pltpu.VMEM((4, CB, D), jnp.float32),
            pltpu.VMEM((4 * CB * L,), jnp.float32),
            pltpu.VMEM_SHARED((N, D), jnp.float32),
        ]
        + [pltpu.SemaphoreType.DMA] * 20
    ),
)
def _msg_kernel(h2_hbm, sl_hbm, radj_hbm, col_hbm, nrep_hbm, out_hbm,
                ri_v, ci_v, rows_v, nexp_v, acc_sh,
                g0, g1, g2, g3, n0, n1, n2, n3, s0, s1, s2, s3,
                i0, i1, i2, i3, c0, c1, c2, c3):
    gs = [g0, g1, g2, g3]
    ns = [n0, n1, n2, n3]
    ss = [s0, s1, s2, s3]
    isem = [i0, i1, i2, i3]
    csem = [c0, c1, c2, c3]
    scid = lax.axis_index("c")
    sid = lax.axis_index("s")
    ebase = sid * ES2
    nb = sid * NROWS

    def issue_ri(c, j, b):
        pltpu.async_copy(
            radj_hbm.at[pl.ds(c * E + ebase + j * CB, CB)], ri_v.at[b], isem[b])

    def wait_ri(b):
        pltpu.make_async_copy(
            radj_hbm.at[pl.ds(0, CB)], ri_v.at[b], isem[b]).wait()

    def issue_ci(j, b):
        pltpu.async_copy(
            col_hbm.at[pl.ds(ebase + j * CB, CB)], ci_v.at[b], csem[b])

    def wait_ci(b):
        pltpu.make_async_copy(
            col_hbm.at[pl.ds(0, CB)], ci_v.at[b], csem[b]).wait()

    def issue_gather(j, b):
        hc = CB // 2
        pltpu.async_copy(h2_hbm.at[ri_v.at[b, pl.ds(0, hc)]],
                         rows_v.at[b, pl.ds(0, hc)], gs[b])
        pltpu.async_copy(h2_hbm.at[ri_v.at[b, pl.ds(hc, hc)]],
                         rows_v.at[b, pl.ds(hc, hc)], gs[b])
        pltpu.async_copy(
            nrep_hbm.at[pl.ds((ebase + j * CB) * L, CB * L)],
            nexp_v.at[pl.ds(b * CB * L, CB * L)], ns[b])

    def wait_gather(b):
        hc = CB // 2
        pltpu.make_async_copy(h2_hbm.at[ri_v.at[b, pl.ds(0, hc)]],
                              rows_v.at[b, pl.ds(0, hc)], gs[b]).wait()
        pltpu.make_async_copy(h2_hbm.at[ri_v.at[b, pl.ds(hc, hc)]],
                              rows_v.at[b, pl.ds(hc, hc)], gs[b]).wait()
        pltpu.make_async_copy(
            nrep_hbm.at[pl.ds(0, CB * L)],
            nexp_v.at[pl.ds(b * CB * L, CB * L)], ns[b]).wait()

    def issue_scatter(b):
        wait_ci(b)
        pltpu.async_copy(
            rows_v.at[b], acc_sh.at[ci_v.at[b]], ss[b], add=True)

    def wait_scatter(b):
        pltpu.make_async_copy(
            rows_v.at[b], acc_sh.at[ci_v.at[b]], ss[b]).wait()

    def compute(b):
        def cgroup(g, _):
            for l in range(L):
                e = g * L + l
                nv = nexp_v[pl.ds(b * CB * L + e * L, L)]
                for q in range(D // L):
                    sl = pl.ds(q * L, L)
                    rows_v[b, e, sl] = rows_v[b, e, sl] * nv
            return 0

        lax.fori_loop(0, CB // L, cgroup, 0)

    def do_channel(k, _):
        c = scid * CPS + k
        pltpu.sync_copy(sl_hbm.at[c, pl.ds(nb, NROWS)],
                        acc_sh.at[pl.ds(nb, NROWS)])

        @pl.when(sid < 2)
        def _():
            rb = NROWS * NSUB + sid * (NREM // 2)
            pltpu.sync_copy(sl_hbm.at[c, pl.ds(rb, NREM // 2)],
                            acc_sh.at[pl.ds(rb, NREM // 2)])

        plsc.subcore_barrier()

        issue_ri(c, 0, 0)
        issue_ri(c, 1, 1)
        issue_ri(c, 2, 2)
        issue_ci(0, 0)
        issue_ci(1, 1)
        wait_ri(0)
        issue_gather(0, 0)
        wait_ri(1)
        issue_gather(1, 1)

        def _switch(fb, b):
            @pl.when(b == 0)
            def _():
                fb(0)

            @pl.when(b == 1)
            def _():
                fb(1)

            @pl.when(b == 2)
            def _():
                fb(2)

            @pl.when(b == 3)
            def _():
                fb(3)

        def stage(j, _):
            @pl.when(j + 2 < NCHK)
            def _():
                bq = lax.rem(j + 2, 4)

                @pl.when(j >= 2)
                def _():
                    _switch(wait_scatter, bq)

                _switch(lambda bb: issue_ci(j + 2, bb), bq)
                _switch(wait_ri, bq)
                _switch(lambda bb: issue_gather(j + 2, bb), bq)

            @pl.when(j + 3 < NCHK)
            def _():
                _switch(lambda bb: issue_ri(c, j + 3, bb), lax.rem(j + 3, 4))

            b = lax.rem(j, 4)
            _switch(wait_gather, b)
            _switch(compute, b)
            _switch(issue_scatter, b)
            return 0

        lax.fori_loop(0, NCHK, stage, 0)
        for bb in range(4):
            wait_scatter(bb)
        plsc.subcore_barrier()
        pltpu.sync_copy(acc_sh.at[pl.ds(nb, NROWS)],
                        out_hbm.at[c, pl.ds(nb, NROWS)])

        @pl.when(sid < 2)
        def _():
            rb = NROWS * NSUB + sid * (NREM // 2)
            pltpu.sync_copy(acc_sh.at[pl.ds(rb, NREM // 2)],
                            out_hbm.at[c, pl.ds(rb, NREM // 2)])

        plsc.subcore_barrier()
        return 0

    lax.fori_loop(0, CPS, do_channel, 0)


def kernel(x0, x1, x2, x3, x4, x5, x6, x7, x8, x9, edge_index, edge_weight, W, b):
    X = jnp.stack([x0, x1, x2, x3, x4, x5, x6, x7, x8, x9])
    row = edge_index[0]
    col = edge_index[1]

    deg_part = _deg_kernel(col, edge_weight)
    dis = _dis_call(deg_part)
    H, SL = _prep_call(X, W, dis[:, None], b)
    norm = _norm_kernel(row, col, edge_weight, dis)
    nrep = _rep_call(norm[:, None])

    radj = _radj_call(row.reshape(E // D, D)).reshape(NCH * E)

    H2 = H.reshape(NCH * N, D)
    OUT = _msg_kernel(H2, SL, radj, col, nrep.reshape(E * L))
    return tuple(OUT[i] for i in range(NCH))


# speedup vs baseline: 4.3411x; 1.0070x over previous
import functools

import jax
import jax.numpy as jnp
from jax import lax
from jax.experimental import pallas as pl
from jax.experimental.pallas import tpu as pltpu
from jax.experimental.pallas import tpu_sc as plsc

N = 10000
E = 320000
D = 128
NCH = 10

NCORES = 2
NSUB = 16
NW = NCORES * NSUB
L = 16
CPS = NCH // NCORES

ES = E // NW
ES2 = E // NSUB
CB = 80
NCHK = ES2 // CB
NROWS = 624
NREM = N - NROWS * NSUB

_mesh = plsc.VectorSubcoreMesh(core_axis_name="c", subcore_axis_name="s")
_sc_params = pltpu.CompilerParams(needs_layout_passes=False)


def _wid():
    return lax.axis_index("s") * NCORES + lax.axis_index("c")


@functools.partial(
    pl.kernel,
    out_type=jax.ShapeDtypeStruct((NW, N), jnp.float32),
    mesh=_mesh,
    compiler_params=_sc_params,
    scratch_types=[
        pltpu.VMEM((N,), jnp.float32),
        pltpu.VMEM((ES,), jnp.int32),
        pltpu.VMEM((ES,), jnp.float32),
    ],
)
def _deg_kernel(col_hbm, ew_hbm, deg_out, deg_v, col_v, ew_v):
    wid = _wid()
    base = wid * ES
    pltpu.sync_copy(col_hbm.at[pl.ds(base, ES)], col_v)
    pltpu.sync_copy(ew_hbm.at[pl.ds(base, ES)], ew_v)

    @plsc.parallel_loop(0, N // L, unroll=4)
    def zbody(i):
        deg_v[pl.ds(i * L, L)] = jnp.zeros((L,), jnp.float32)

    def ebody(i, _):
        sl = pl.ds(i * L, L)
        plsc.addupdate_scatter(deg_v, [col_v[sl]], ew_v[sl])
        return 0

    lax.fori_loop(0, ES // L, ebody, 0)
    pltpu.sync_copy(deg_v, deg_out.at[wid])


def _prep_body(degp_ref, x_ref, w_ref, b_ref, row_ref,
               dis_ref, h_ref, sl_ref, radj_ref):
    deg = jnp.sum(degp_ref[...], axis=0, keepdims=True) + 1.0
    dis = jnp.where(deg > 0, lax.rsqrt(jnp.maximum(deg, 1e-12)), 0.0)

    @pl.when(pl.program_id(0) == 0)
    def _():
        dis_ref[...] = dis.reshape(N)

    h = lax.dot_general(
        x_ref[0], w_ref[0],
        dimension_numbers=(((1,), (0,)), ((), ())),
        preferred_element_type=jnp.float32,
    )
    h_ref[0] = h
    d2 = (dis * dis).reshape(N, 1)
    sl_ref[0] = d2 * h + b_ref[pl.program_id(0)][None, :]
    radj_ref[0] = row_ref[...] + pl.program_id(0) * N


def _prep_call(deg_part, X, W, b, row2):
    nr = E // D
    return pl.pallas_call(
        _prep_body,
        grid=(NCH,),
        in_specs=[
            pl.BlockSpec((NW, N), lambda i: (0, 0)),
            pl.BlockSpec((1, N, D), lambda i: (i, 0, 0)),
            pl.BlockSpec((1, D, D), lambda i: (i, 0, 0)),
            pl.BlockSpec((NCH, D), lambda i: (0, 0)),
            pl.BlockSpec((nr, D), lambda i: (0, 0)),
        ],
        out_specs=(
            pl.BlockSpec((N,), lambda i: (0,)),
            pl.BlockSpec((1, N, D), lambda i: (i, 0, 0)),
            pl.BlockSpec((1, N, D), lambda i: (i, 0, 0)),
            pl.BlockSpec((1, nr, D), lambda i: (i, 0, 0)),
        ),
        out_shape=(
            jax.ShapeDtypeStruct((N,), jnp.float32),
            jax.ShapeDtypeStruct((NCH, N, D), jnp.float32),
            jax.ShapeDtypeStruct((NCH, N, D), jnp.float32),
            jax.ShapeDtypeStruct((NCH, E // D, D), jnp.int32),
        ),
    )(deg_part, X, W, b, row2)


@functools.partial(
    pl.kernel,
    out_type=jax.ShapeDtypeStruct((E,), jnp.float32),
    mesh=_mesh,
    compiler_params=_sc_params,
    scratch_types=[
        pltpu.VMEM((N,), jnp.float32),
        pltpu.VMEM((ES,), jnp.int32),
        pltpu.VMEM((ES,), jnp.int32),
        pltpu.VMEM((ES,), jnp.float32),
        pltpu.VMEM((ES,), jnp.float32),
    ],
)
def _norm_kernel(row_hbm, col_hbm, ew_hbm, dis_hbm, norm_out,
                 dis_v, row_v, col_v, ew_v, norm_v):
    wid = _wid()
    base = wid * ES
    pltpu.sync_copy(dis_hbm, dis_v)
    pltpu.sync_copy(row_hbm.at[pl.ds(base, ES)], row_v)
    pltpu.sync_copy(col_hbm.at[pl.ds(base, ES)], col_v)
    pltpu.sync_copy(ew_hbm.at[pl.ds(base, ES)], ew_v)

    def body(i, _):
        sl = pl.ds(i * L, L)
        dr = plsc.load_gather(dis_v, [row_v[sl]])
        dc = plsc.load_gather(dis_v, [col_v[sl]])
        norm_v[sl] = dr * ew_v[sl] * dc
        return 0

    lax.fori_loop(0, ES // L, body, 0)
    pltpu.sync_copy(norm_v, norm_out.at[pl.ds(base, ES)])


def _rep_body(n_ref, o_ref):
    o_ref[...] = n_ref[...] * jnp.ones((1, L), jnp.float32)


def _rep_call(norm2):
    blk = E // 40
    return pl.pallas_call(
        _rep_body,
        grid=(40,),
        in_specs=[pl.BlockSpec((blk, 1), lambda i: (i, 0))],
        out_specs=pl.BlockSpec((blk, L), lambda i: (i, 0)),
        out_shape=jax.ShapeDtypeStruct((E, L), jnp.float32),
    )(norm2)


@functools.partial(
    pl.kernel,
    out_type=jax.ShapeDtypeStruct((NCH, N, D), jnp.float32),
    mesh=_mesh,
    compiler_params=_sc_params,
    scratch_types=(
        [
            pltpu.VMEM((4, CB), jnp.int32),
            pltpu.VMEM((4, CB), jnp.int32),
            pltpu.VMEM((4, CB, D), jnp.float32),
            pltpu.VMEM((4 * CB * L,), jnp.float32),
            pltpu.VMEM_SHARED((N, D), jnp.float32),
        ]
        + [pltpu.SemaphoreType.DMA] * 20
    ),
)
def _msg_kernel(h2_hbm, sl_hbm, radj_hbm, col_hbm, nrep_hbm, out_hbm,
                ri_v, ci_v, rows_v, nexp_v, acc_sh,
                g0, g1, g2, g3, n0, n1, n2, n3, s0, s1, s2, s3,
                i0, i1, i2, i3, c0, c1, c2, c3):
    gs = [g0, g1, g2, g3]
    ns = [n0, n1, n2, n3]
    ss = [s0, s1, s2, s3]
    isem = [i0, i1, i2, i3]
    csem = [c0, c1, c2, c3]
    scid = lax.axis_index("c")
    sid = lax.axis_index("s")
    ebase = sid * ES2
    nb = sid * NROWS

    def issue_ri(c, j, b):
        pltpu.async_copy(
            radj_hbm.at[pl.ds(c * E + ebase + j * CB, CB)], ri_v.at[b], isem[b])

    def wait_ri(b):
        pltpu.make_async_copy(
            radj_hbm.at[pl.ds(0, CB)], ri_v.at[b], isem[b]).wait()

    def issue_ci(j, b):
        pltpu.async_copy(
            col_hbm.at[pl.ds(ebase + j * CB, CB)], ci_v.at[b], csem[b])

    def wait_ci(b):
        pltpu.make_async_copy(
            col_hbm.at[pl.ds(0, CB)], ci_v.at[b], csem[b]).wait()

    def issue_gather(j, b):
        pltpu.async_copy(h2_hbm.at[ri_v.at[b]], rows_v.at[b], gs[b])
        pltpu.async_copy(
            nrep_hbm.at[pl.ds((ebase + j * CB) * L, CB * L)],
            nexp_v.at[pl.ds(b * CB * L, CB * L)], ns[b])

    def wait_gather(b):
        pltpu.make_async_copy(
            h2_hbm.at[ri_v.at[b]], rows_v.at[b], gs[b]).wait()
        pltpu.make_async_copy(
            nrep_hbm.at[pl.ds(0, CB * L)],
            nexp_v.at[pl.ds(b * CB * L, CB * L)], ns[b]).wait()

    def issue_scatter(b):
        wait_ci(b)
        pltpu.async_copy(
            rows_v.at[b], acc_sh.at[ci_v.at[b]], ss[b], add=True)

    def wait_scatter(b):
        pltpu.make_async_copy(
            rows_v.at[b], acc_sh.at[ci_v.at[b]], ss[b]).wait()

    def compute(b):
        def cgroup(g, _):
            for l in range(L):
                e = g * L + l
                nv = nexp_v[pl.ds(b * CB * L + e * L, L)]
                for q in range(D // L):
                    sl = pl.ds(q * L, L)
                    rows_v[b, e, sl] = rows_v[b, e, sl] * nv
            return 0

        lax.fori_loop(0, CB // L, cgroup, 0)

    def do_channel(k, _):
        c = scid * CPS + k
        pltpu.sync_copy(sl_hbm.at[c, pl.ds(nb, NROWS)],
                        acc_sh.at[pl.ds(nb, NROWS)])

        @pl.when(sid < 2)
        def _():
            rb = NROWS * NSUB + sid * (NREM // 2)
            pltpu.sync_copy(sl_hbm.at[c, pl.ds(rb, NREM // 2)],
                            acc_sh.at[pl.ds(rb, NREM // 2)])

        plsc.subcore_barrier()

        issue_ri(c, 0, 0)
        issue_ri(c, 1, 1)
        issue_ri(c, 2, 2)
        issue_ci(0, 0)
        issue_ci(1, 1)
        wait_ri(0)
        issue_gather(0, 0)
        wait_ri(1)
        issue_gather(1, 1)

        def _switch(fb, b):
            @pl.when(b == 0)
            def _():
                fb(0)

            @pl.when(b == 1)
            def _():
                fb(1)

            @pl.when(b == 2)
            def _():
                fb(2)

            @pl.when(b == 3)
            def _():
                fb(3)

        def stage(j, _):
            @pl.when(j + 2 < NCHK)
            def _():
                bq = lax.rem(j + 2, 4)

                @pl.when(j >= 2)
                def _():
                    _switch(wait_scatter, bq)

                _switch(lambda bb: issue_ci(j + 2, bb), bq)
                _switch(wait_ri, bq)
                _switch(lambda bb: issue_gather(j + 2, bb), bq)

            @pl.when(j + 3 < NCHK)
            def _():
                _switch(lambda bb: issue_ri(c, j + 3, bb), lax.rem(j + 3, 4))

            b = lax.rem(j, 4)
            _switch(wait_gather, b)
            _switch(compute, b)
            _switch(issue_scatter, b)
            return 0

        lax.fori_loop(0, NCHK, stage, 0)
        for bb in range(4):
            wait_scatter(bb)
        plsc.subcore_barrier()
        pltpu.sync_copy(acc_sh.at[pl.ds(nb, NROWS)],
                        out_hbm.at[c, pl.ds(nb, NROWS)])

        @pl.when(sid < 2)
        def _():
            rb = NROWS * NSUB + sid * (NREM // 2)
            pltpu.sync_copy(acc_sh.at[pl.ds(rb, NREM // 2)],
                            out_hbm.at[c, pl.ds(rb, NREM // 2)])

        plsc.subcore_barrier()
        return 0

    lax.fori_loop(0, CPS, do_channel, 0)


def kernel(x0, x1, x2, x3, x4, x5, x6, x7, x8, x9, edge_index, edge_weight, W, b):
    X = jnp.stack([x0, x1, x2, x3, x4, x5, x6, x7, x8, x9])
    row = edge_index[0]
    col = edge_index[1]

    deg_part = _deg_kernel(col, edge_weight)
    dis, H, SL, radj = _prep_call(deg_part, X, W, b, row.reshape(E // D, D))
    radj = radj.reshape(NCH * E)
    norm = _norm_kernel(row, col, edge_weight, dis)
    nrep = _rep_call(norm[:, None])

    H2 = H.reshape(NCH * N, D)
    OUT = _msg_kernel(H2, SL, radj, col, nrep.reshape(E * L))
    return tuple(OUT[i] for i in range(NCH))


# speedup vs baseline: 5.5700x; 1.2831x over previous
import functools

import jax
import jax.numpy as jnp
from jax import lax
from jax.experimental import pallas as pl
from jax.experimental.pallas import tpu as pltpu
from jax.experimental.pallas import tpu_sc as plsc

N = 10000
E = 320000
D = 128
NCH = 10

NCORES = 2
NSUB = 16
NW = NCORES * NSUB
L = 16
CPS = NCH // NCORES

ES = E // NW
ES2 = E // NSUB
CB = 80
NCHK = ES2 // CB
NROWS = 624
NREM = N - NROWS * NSUB

_mesh = plsc.VectorSubcoreMesh(core_axis_name="c", subcore_axis_name="s")
_sc_params = pltpu.CompilerParams(needs_layout_passes=False)


def _wid():
    return lax.axis_index("s") * NCORES + lax.axis_index("c")


@functools.partial(
    pl.kernel,
    out_type=jax.ShapeDtypeStruct((NW, N), jnp.float32),
    mesh=_mesh,
    compiler_params=_sc_params,
    scratch_types=[
        pltpu.VMEM((N,), jnp.float32),
        pltpu.VMEM((ES,), jnp.int32),
        pltpu.VMEM((ES,), jnp.float32),
    ],
)
def _deg_kernel(col_hbm, ew_hbm, deg_out, deg_v, col_v, ew_v):
    wid = _wid()
    base = wid * ES
    pltpu.sync_copy(col_hbm.at[pl.ds(base, ES)], col_v)
    pltpu.sync_copy(ew_hbm.at[pl.ds(base, ES)], ew_v)

    @plsc.parallel_loop(0, N // L, unroll=4)
    def zbody(i):
        deg_v[pl.ds(i * L, L)] = jnp.zeros((L,), jnp.float32)

    def ebody(i, _):
        sl = pl.ds(i * L, L)
        plsc.addupdate_scatter(deg_v, [col_v[sl]], ew_v[sl])
        return 0

    lax.fori_loop(0, ES // L, ebody, 0)
    pltpu.sync_copy(deg_v, deg_out.at[wid])


def _prep_body(degp_ref, x_ref, w_ref, b_ref, row_ref,
               dis_ref, h_ref, sl_ref, radj_ref):
    deg = jnp.sum(degp_ref[...], axis=0, keepdims=True) + 1.0
    dis = jnp.where(deg > 0, lax.rsqrt(jnp.maximum(deg, 1e-12)), 0.0)

    @pl.when(pl.program_id(0) == 0)
    def _():
        dis_ref[...] = dis.reshape(N)

    h = lax.dot_general(
        x_ref[0], w_ref[0],
        dimension_numbers=(((1,), (0,)), ((), ())),
        preferred_element_type=jnp.float32,
    )
    h_ref[0] = h
    d2 = (dis * dis).reshape(N, 1)
    sl_ref[0] = d2 * h + b_ref[pl.program_id(0)][None, :]
    radj_ref[0] = row_ref[...] + pl.program_id(0) * N


def _prep_call(deg_part, X, W, b, row2):
    nr = E // D
    return pl.pallas_call(
        _prep_body,
        grid=(NCH,),
        in_specs=[
            pl.BlockSpec((NW, N), lambda i: (0, 0)),
            pl.BlockSpec((1, N, D), lambda i: (i, 0, 0)),
            pl.BlockSpec((1, D, D), lambda i: (i, 0, 0)),
            pl.BlockSpec((NCH, D), lambda i: (0, 0)),
            pl.BlockSpec((nr, D), lambda i: (0, 0)),
        ],
        out_specs=(
            pl.BlockSpec((N,), lambda i: (0,)),
            pl.BlockSpec((1, N, D), lambda i: (i, 0, 0)),
            pl.BlockSpec((1, N, D), lambda i: (i, 0, 0)),
            pl.BlockSpec((1, nr, D), lambda i: (i, 0, 0)),
        ),
        out_shape=(
            jax.ShapeDtypeStruct((N,), jnp.float32),
            jax.ShapeDtypeStruct((NCH, N, D), jnp.float32),
            jax.ShapeDtypeStruct((NCH, N, D), jnp.float32),
            jax.ShapeDtypeStruct((NCH, E // D, D), jnp.int32),
        ),
    )(deg_part, X, W, b, row2)


@functools.partial(
    pl.kernel,
    out_type=jax.ShapeDtypeStruct((E,), jnp.float32),
    mesh=_mesh,
    compiler_params=_sc_params,
    scratch_types=[
        pltpu.VMEM((N,), jnp.float32),
        pltpu.VMEM((ES,), jnp.int32),
        pltpu.VMEM((ES,), jnp.int32),
        pltpu.VMEM((ES,), jnp.float32),
        pltpu.VMEM((ES,), jnp.float32),
    ],
)
def _norm_kernel(row_hbm, col_hbm, ew_hbm, dis_hbm, norm_out,
                 dis_v, row_v, col_v, ew_v, norm_v):
    wid = _wid()
    base = wid * ES
    pltpu.sync_copy(dis_hbm, dis_v)
    pltpu.sync_copy(row_hbm.at[pl.ds(base, ES)], row_v)
    pltpu.sync_copy(col_hbm.at[pl.ds(base, ES)], col_v)
    pltpu.sync_copy(ew_hbm.at[pl.ds(base, ES)], ew_v)

    def body(i, _):
        sl = pl.ds(i * L, L)
        dr = plsc.load_gather(dis_v, [row_v[sl]])
        dc = plsc.load_gather(dis_v, [col_v[sl]])
        norm_v[sl] = dr * ew_v[sl] * dc
        return 0

    lax.fori_loop(0, ES // L, body, 0)
    pltpu.sync_copy(norm_v, norm_out.at[pl.ds(base, ES)])


def _rep_body(n_ref, o_ref):
    o_ref[...] = n_ref[...] * jnp.ones((1, L), jnp.float32)


def _rep_call(norm2):
    blk = E // 40
    return pl.pallas_call(
        _rep_body,
        grid=(40,),
        in_specs=[pl.BlockSpec((blk, 1), lambda i: (i, 0))],
        out_specs=pl.BlockSpec((blk, L), lambda i: (i, 0)),
        out_shape=jax.ShapeDtypeStruct((E, L), jnp.float32),
    )(norm2)


@functools.partial(
    pl.kernel,
    out_type=jax.ShapeDtypeStruct((NCH, N, D), jnp.float32),
    mesh=_mesh,
    compiler_params=_sc_params,
    scratch_types=(
        [
            pltpu.VMEM((4, CB), jnp.int32),
            pltpu.VMEM((4, CB), jnp.int32),
            pltpu.VMEM((4, CB, D), jnp.float32),
            pltpu.VMEM((4 * CB,), jnp.float32),
            pltpu.VMEM_SHARED((N, D), jnp.float32),
        ]
        + [pltpu.SemaphoreType.DMA] * 20
    ),
)
def _msg_kernel(h2_hbm, sl_hbm, radj_hbm, col_hbm, norm_hbm, out_hbm,
                ri_v, ci_v, rows_v, nexp_v, acc_sh,
                g0, g1, g2, g3, n0, n1, n2, n3, s0, s1, s2, s3,
                i0, i1, i2, i3, c0, c1, c2, c3):
    gs = [g0, g1, g2, g3]
    ns = [n0, n1, n2, n3]
    ss = [s0, s1, s2, s3]
    isem = [i0, i1, i2, i3]
    csem = [c0, c1, c2, c3]
    scid = lax.axis_index("c")
    sid = lax.axis_index("s")
    ebase = sid * ES2
    nb = sid * NROWS

    def issue_ri(c, j, b):
        pltpu.async_copy(
            radj_hbm.at[pl.ds(c * E + ebase + j * CB, CB)], ri_v.at[b], isem[b])

    def wait_ri(b):
        pltpu.make_async_copy(
            radj_hbm.at[pl.ds(0, CB)], ri_v.at[b], isem[b]).wait()

    def issue_ci(j, b):
        pltpu.async_copy(
            col_hbm.at[pl.ds(ebase + j * CB, CB)], ci_v.at[b], csem[b])

    def wait_ci(b):
        pltpu.make_async_copy(
            col_hbm.at[pl.ds(0, CB)], ci_v.at[b], csem[b]).wait()

    def issue_gather(j, b):
        pltpu.async_copy(h2_hbm.at[ri_v.at[b]], rows_v.at[b], gs[b])
        pltpu.async_copy(
            norm_hbm.at[pl.ds(ebase + j * CB, CB)],
            nexp_v.at[pl.ds(b * CB, CB)], ns[b])

    def wait_gather(b):
        pltpu.make_async_copy(
            h2_hbm.at[ri_v.at[b]], rows_v.at[b], gs[b]).wait()
        pltpu.make_async_copy(
            norm_hbm.at[pl.ds(0, CB)],
            nexp_v.at[pl.ds(b * CB, CB)], ns[b]).wait()

    def issue_scatter(b):
        wait_ci(b)
        pltpu.async_copy(
            rows_v.at[b], acc_sh.at[ci_v.at[b]], ss[b], add=True)

    def wait_scatter(b):
        pltpu.make_async_copy(
            rows_v.at[b], acc_sh.at[ci_v.at[b]], ss[b]).wait()

    def compute(b):
        def cgroup(g, _):
            for l in range(L):
                e = g * L + l
                nv = plsc.load_gather(
                    nexp_v, [jnp.full((L,), b * CB + e, jnp.int32)])
                for q in range(D // L):
                    sl = pl.ds(q * L, L)
                    rows_v[b, e, sl] = rows_v[b, e, sl] * nv
            return 0

        lax.fori_loop(0, CB // L, cgroup, 0)

    def do_channel(k, _):
        c = scid * CPS + k
        pltpu.sync_copy(sl_hbm.at[c, pl.ds(nb, NROWS)],
                        acc_sh.at[pl.ds(nb, NROWS)])

        @pl.when(sid < 2)
        def _():
            rb = NROWS * NSUB + sid * (NREM // 2)
            pltpu.sync_copy(sl_hbm.at[c, pl.ds(rb, NREM // 2)],
                            acc_sh.at[pl.ds(rb, NREM // 2)])

        plsc.subcore_barrier()

        issue_ri(c, 0, 0)
        issue_ri(c, 1, 1)
        issue_ri(c, 2, 2)
        issue_ci(0, 0)
        issue_ci(1, 1)
        wait_ri(0)
        issue_gather(0, 0)
        wait_ri(1)
        issue_gather(1, 1)

        def _switch(fb, b):
            @pl.when(b == 0)
            def _():
                fb(0)

            @pl.when(b == 1)
            def _():
                fb(1)

            @pl.when(b == 2)
            def _():
                fb(2)

            @pl.when(b == 3)
            def _():
                fb(3)

        def stage(j, _):
            @pl.when(j + 2 < NCHK)
            def _():
                bq = lax.rem(j + 2, 4)

                @pl.when(j >= 2)
                def _():
                    _switch(wait_scatter, bq)

                _switch(lambda bb: issue_ci(j + 2, bb), bq)
                _switch(wait_ri, bq)
                _switch(lambda bb: issue_gather(j + 2, bb), bq)

            @pl.when(j + 3 < NCHK)
            def _():
                _switch(lambda bb: issue_ri(c, j + 3, bb), lax.rem(j + 3, 4))

            b = lax.rem(j, 4)
            _switch(wait_gather, b)
            _switch(compute, b)
            _switch(issue_scatter, b)
            return 0

        lax.fori_loop(0, NCHK, stage, 0)
        for bb in range(4):
            wait_scatter(bb)
        plsc.subcore_barrier()
        pltpu.sync_copy(acc_sh.at[pl.ds(nb, NROWS)],
                        out_hbm.at[c, pl.ds(nb, NROWS)])

        @pl.when(sid < 2)
        def _():
            rb = NROWS * NSUB + sid * (NREM // 2)
            pltpu.sync_copy(acc_sh.at[pl.ds(rb, NREM // 2)],
                            out_hbm.at[c, pl.ds(rb, NREM // 2)])

        plsc.subcore_barrier()
        return 0

    lax.fori_loop(0, CPS, do_channel, 0)


def kernel(x0, x1, x2, x3, x4, x5, x6, x7, x8, x9, edge_index, edge_weight, W, b):
    X = jnp.stack([x0, x1, x2, x3, x4, x5, x6, x7, x8, x9])
    row = edge_index[0]
    col = edge_index[1]

    deg_part = _deg_kernel(col, edge_weight)
    dis, H, SL, radj = _prep_call(deg_part, X, W, b, row.reshape(E // D, D))
    radj = radj.reshape(NCH * E)
    norm = _norm_kernel(row, col, edge_weight, dis)

    H2 = H.reshape(NCH * N, D)
    OUT = _msg_kernel(H2, SL, radj, col, norm)
    return tuple(OUT[i] for i in range(NCH))


# speedup vs baseline: 5.5703x; 1.0000x over previous
import functools

import jax
import jax.numpy as jnp
from jax import lax
from jax.experimental import pallas as pl
from jax.experimental.pallas import tpu as pltpu
from jax.experimental.pallas import tpu_sc as plsc

N = 10000
E = 320000
D = 128
NCH = 10

NCORES = 2
NSUB = 16
NW = NCORES * NSUB
L = 16
CPS = NCH // NCORES

ES = E // NW
ES2 = E // NSUB
CB = 80
NCHK = ES2 // CB
NROWS = 624
NREM = N - NROWS * NSUB

_mesh = plsc.VectorSubcoreMesh(core_axis_name="c", subcore_axis_name="s")
_sc_params = pltpu.CompilerParams(needs_layout_passes=False)


def _wid():
    return lax.axis_index("s") * NCORES + lax.axis_index("c")


@functools.partial(
    pl.kernel,
    out_type=jax.ShapeDtypeStruct((NW, N), jnp.float32),
    mesh=_mesh,
    compiler_params=_sc_params,
    scratch_types=[
        pltpu.VMEM((N,), jnp.float32),
        pltpu.VMEM((ES,), jnp.int32),
        pltpu.VMEM((ES,), jnp.float32),
    ],
)
def _deg_kernel(col_hbm, ew_hbm, deg_out, deg_v, col_v, ew_v):
    wid = _wid()
    base = wid * ES
    pltpu.sync_copy(col_hbm.at[pl.ds(base, ES)], col_v)
    pltpu.sync_copy(ew_hbm.at[pl.ds(base, ES)], ew_v)

    @plsc.parallel_loop(0, N // L, unroll=4)
    def zbody(i):
        deg_v[pl.ds(i * L, L)] = jnp.zeros((L,), jnp.float32)

    def ebody(i, _):
        sl = pl.ds(i * L, L)
        plsc.addupdate_scatter(deg_v, [col_v[sl]], ew_v[sl])
        return 0

    lax.fori_loop(0, ES // L, ebody, 0)
    pltpu.sync_copy(deg_v, deg_out.at[wid])


def _prep_body(degp_ref, x_ref, w_ref, b_ref, row_ref,
               dis_ref, h_ref, sl_ref, radj_ref):
    deg = jnp.sum(degp_ref[...], axis=0, keepdims=True) + 1.0
    dis = jnp.where(deg > 0, lax.rsqrt(jnp.maximum(deg, 1e-12)), 0.0)

    @pl.when(pl.program_id(0) == 0)
    def _():
        dis_ref[...] = dis.reshape(N)

    h = lax.dot_general(
        x_ref[0], w_ref[0],
        dimension_numbers=(((1,), (0,)), ((), ())),
        preferred_element_type=jnp.float32,
    )
    h_ref[0] = h
    d2 = (dis * dis).reshape(N, 1)
    sl_ref[0] = d2 * h + b_ref[pl.program_id(0)][None, :]
    radj_ref[0] = row_ref[...] + pl.program_id(0) * N


def _prep_call(deg_part, X, W, b, row2):
    nr = E // D
    return pl.pallas_call(
        _prep_body,
        grid=(NCH,),
        in_specs=[
            pl.BlockSpec((NW, N), lambda i: (0, 0)),
            pl.BlockSpec((1, N, D), lambda i: (i, 0, 0)),
            pl.BlockSpec((1, D, D), lambda i: (i, 0, 0)),
            pl.BlockSpec((NCH, D), lambda i: (0, 0)),
            pl.BlockSpec((nr, D), lambda i: (0, 0)),
        ],
        out_specs=(
            pl.BlockSpec((N,), lambda i: (0,)),
            pl.BlockSpec((1, N, D), lambda i: (i, 0, 0)),
            pl.BlockSpec((1, N, D), lambda i: (i, 0, 0)),
            pl.BlockSpec((1, nr, D), lambda i: (i, 0, 0)),
        ),
        out_shape=(
            jax.ShapeDtypeStruct((N,), jnp.float32),
            jax.ShapeDtypeStruct((NCH, N, D), jnp.float32),
            jax.ShapeDtypeStruct((NCH, N, D), jnp.float32),
            jax.ShapeDtypeStruct((NCH, E // D, D), jnp.int32),
        ),
    )(deg_part, X, W, b, row2)


@functools.partial(
    pl.kernel,
    out_type=jax.ShapeDtypeStruct((E,), jnp.float32),
    mesh=_mesh,
    compiler_params=_sc_params,
    scratch_types=[
        pltpu.VMEM((N,), jnp.float32),
        pltpu.VMEM((ES,), jnp.int32),
        pltpu.VMEM((ES,), jnp.int32),
        pltpu.VMEM((ES,), jnp.float32),
        pltpu.VMEM((ES,), jnp.float32),
    ],
)
def _norm_kernel(row_hbm, col_hbm, ew_hbm, dis_hbm, norm_out,
                 dis_v, row_v, col_v, ew_v, norm_v):
    wid = _wid()
    base = wid * ES
    pltpu.sync_copy(dis_hbm, dis_v)
    pltpu.sync_copy(row_hbm.at[pl.ds(base, ES)], row_v)
    pltpu.sync_copy(col_hbm.at[pl.ds(base, ES)], col_v)
    pltpu.sync_copy(ew_hbm.at[pl.ds(base, ES)], ew_v)

    def body(i, _):
        sl = pl.ds(i * L, L)
        dr = plsc.load_gather(dis_v, [row_v[sl]])
        dc = plsc.load_gather(dis_v, [col_v[sl]])
        norm_v[sl] = dr * ew_v[sl] * dc
        return 0

    lax.fori_loop(0, ES // L, body, 0)
    pltpu.sync_copy(norm_v, norm_out.at[pl.ds(base, ES)])


@functools.partial(
    pl.kernel,
    out_type=jax.ShapeDtypeStruct((NCH, N, D), jnp.float32),
    mesh=_mesh,
    compiler_params=_sc_params,
    scratch_types=(
        [
            pltpu.VMEM((4, CB), jnp.int32),
            pltpu.VMEM((4, CB), jnp.int32),
            pltpu.VMEM((4, CB, D), jnp.float32),
            pltpu.VMEM((4 * CB,), jnp.float32),
            pltpu.VMEM_SHARED((N, D), jnp.float32),
        ]
        + [pltpu.SemaphoreType.DMA] * 20
    ),
)
def _msg_kernel(h2_hbm, sl_hbm, radj_hbm, col_hbm, norm_hbm, out_hbm,
                ri_v, ci_v, rows_v, nexp_v, acc_sh,
                g0, g1, g2, g3, n0, n1, n2, n3, s0, s1, s2, s3,
                i0, i1, i2, i3, c0, c1, c2, c3):
    gs = [g0, g1, g2, g3]
    ns = [n0, n1, n2, n3]
    ss = [s0, s1, s2, s3]
    isem = [i0, i1, i2, i3]
    csem = [c0, c1, c2, c3]
    scid = lax.axis_index("c")
    sid = lax.axis_index("s")
    ebase = sid * ES2
    nb = sid * NROWS

    def issue_ri(c, j, b):
        pltpu.async_copy(
            radj_hbm.at[pl.ds(c * E + ebase + j * CB, CB)], ri_v.at[b], isem[b])

    def wait_ri(b):
        pltpu.make_async_copy(
            radj_hbm.at[pl.ds(0, CB)], ri_v.at[b], isem[b]).wait()

    def issue_ci(j, b):
        pltpu.async_copy(
            col_hbm.at[pl.ds(ebase + j * CB, CB)], ci_v.at[b], csem[b])

    def wait_ci(b):
        pltpu.make_async_copy(
            col_hbm.at[pl.ds(0, CB)], ci_v.at[b], csem[b]).wait()

    def issue_gather(j, b):
        pltpu.async_copy(h2_hbm.at[ri_v.at[b]], rows_v.at[b], gs[b])
        pltpu.async_copy(
            norm_hbm.at[pl.ds(ebase + j * CB, CB)],
            nexp_v.at[pl.ds(b * CB, CB)], ns[b])

    def wait_gather(b):
        pltpu.make_async_copy(
            h2_hbm.at[ri_v.at[b]], rows_v.at[b], gs[b]).wait()
        pltpu.make_async_copy(
            norm_hbm.at[pl.ds(0, CB)],
            nexp_v.at[pl.ds(b * CB, CB)], ns[b]).wait()

    def issue_scatter(b):
        wait_ci(b)
        pltpu.async_copy(
            rows_v.at[b], acc_sh.at[ci_v.at[b]], ss[b], add=True)

    def wait_scatter(b):
        pltpu.make_async_copy(
            rows_v.at[b], acc_sh.at[ci_v.at[b]], ss[b]).wait()

    def compute(b):
        def cgroup(g, _):
            for l in range(L):
                e = g * L + l
                nv = plsc.load_gather(
                    nexp_v, [jnp.full((L,), b * CB + e, jnp.int32)])
                for q in range(D // L):
                    sl = pl.ds(q * L, L)
                    rows_v[b, e, sl] = rows_v[b, e, sl] * nv
            return 0

        lax.fori_loop(0, CB // L, cgroup, 0)

    def do_channel(k, _):
        c = scid * CPS + k
        pltpu.sync_copy(sl_hbm.at[c, pl.ds(nb, NROWS)],
                        acc_sh.at[pl.ds(nb, NROWS)])

        @pl.when(sid < 2)
        def _():
            rb = NROWS * NSUB + sid * (NREM // 2)
            pltpu.sync_copy(sl_hbm.at[c, pl.ds(rb, NREM // 2)],
                            acc_sh.at[pl.ds(rb, NREM // 2)])

        plsc.subcore_barrier()

        issue_ri(c, 0, 0)
        issue_ri(c, 1, 1)
        issue_ri(c, 2, 2)
        issue_ci(0, 0)
        issue_ci(1, 1)
        wait_ri(0)
        issue_gather(0, 0)
        wait_ri(1)
        issue_gather(1, 1)

        def _switch(fb, b):
            @pl.when(b == 0)
            def _():
                fb(0)

            @pl.when(b == 1)
            def _():
                fb(1)

            @pl.when(b == 2)
            def _():
                fb(2)

            @pl.when(b == 3)
            def _():
                fb(3)

        def stage(j, _):
            @pl.when(j + 2 < NCHK)
            def _():
                bq = lax.rem(j + 2, 4)

                @pl.when(j >= 2)
                def _():
                    _switch(wait_scatter, bq)

                _switch(lambda bb: issue_ci(j + 2, bb), bq)
                _switch(wait_ri, bq)
                _switch(lambda bb: issue_gather(j + 2, bb), bq)

            @pl.when(j + 3 < NCHK)
            def _():
                _switch(lambda bb: issue_ri(c, j + 3, bb), lax.rem(j + 3, 4))

            b = lax.rem(j, 4)
            _switch(wait_gather, b)
            _switch(compute, b)
            _switch(issue_scatter, b)
            return 0

        lax.fori_loop(0, NCHK, stage, 0)
        for bb in range(4):
            wait_scatter(bb)
        plsc.subcore_barrier()
        pltpu.sync_copy(acc_sh.at[pl.ds(nb, NROWS)],
                        out_hbm.at[c, pl.ds(nb, NROWS)])

        @pl.when(sid < 2)
        def _():
            rb = NROWS * NSUB + sid * (NREM // 2)
            pltpu.sync_copy(acc_sh.at[pl.ds(rb, NREM // 2)],
                            out_hbm.at[c, pl.ds(rb, NREM // 2)])

        plsc.subcore_barrier()
        return 0

    lax.fori_loop(0, CPS, do_channel, 0)


def kernel(x0, x1, x2, x3, x4, x5, x6, x7, x8, x9, edge_index, edge_weight, W, b):
    X = jnp.stack([x0, x1, x2, x3, x4, x5, x6, x7, x8, x9])
    row = edge_index[0]
    col = edge_index[1]

    deg_part = _deg_kernel(col, edge_weight)
    dis, H, SL, radj = _prep_call(deg_part, X, W, b, row.reshape(E // D, D))
    radj = radj.reshape(NCH * E)
    norm = _norm_kernel(row, col, edge_weight, dis)

    H2 = H.reshape(NCH * N, D)
    OUT = _msg_kernel(H2, SL, radj, col, norm)
    return tuple(OUT[i] for i in range(NCH))
